# baseline (speedup 1.0000x reference)
import functools

import jax
import jax.numpy as jnp
from jax import lax
from jax.experimental import pallas as pl
from jax.experimental.pallas import tpu as pltpu
from jax.experimental.pallas import tpu_sc as plsc

NUM_EMB = 1000000
D = 64
B = 4096
BAG = 50

NC = 2
NS = 16
NW = NC * NS

BAGS_PER_W = B // NW
IDX_PER_W = BAGS_PER_W * BAG
CB = 8
CHUNK_IDX = CB * BAG
N_CHUNKS = BAGS_PER_W // CB
GATHER = 80
N_GATHER = CHUNK_IDX // GATHER


def _ebag_body(idx_hbm, table_hbm, out_hbm, idx_v, rows0, rows1, out_v, sem0, sem1):
    wid = lax.axis_index("s") * NC + lax.axis_index("c")
    pltpu.sync_copy(idx_hbm.at[pl.ds(wid * IDX_PER_W, IDX_PER_W)], idx_v)

    bufs = (rows0, rows1)
    sems = (sem0, sem1)

    def fire(c, buf, sem):
        base = c * CHUNK_IDX
        for j in range(N_GATHER):
            pltpu.async_copy(
                table_hbm.at[idx_v.at[pl.ds(base + j * GATHER, GATHER)]],
                buf.at[pl.ds(j * GATHER, GATHER)],
                sem,
            )

    def drain(buf, sem):
        for j in range(N_GATHER):
            pltpu.make_async_copy(
                table_hbm.at[idx_v.at[pl.ds(j * GATHER, GATHER)]],
                buf.at[pl.ds(j * GATHER, GATHER)],
                sem,
            ).wait()

    def accumulate(c, buf):
        def bag_body(b, carry):
            row0 = b * BAG
            accs = [jnp.zeros((16,), jnp.float32) for _ in range(4)]
            for r in range(BAG):
                for k in range(4):
                    accs[k] = accs[k] + buf[row0 + r, pl.ds(k * 16, 16)]
            for k in range(4):
                out_v[c * CB + b, pl.ds(k * 16, 16)] = accs[k] * jnp.float32(1.0 / BAG)
            return carry

        lax.fori_loop(0, CB, bag_body, 0)

    fire(0, rows0, sem0)
    fire(1, rows1, sem1)

    def chunk_body(u, carry):
        for j in range(2):
            c = 2 * u + j
            drain(bufs[j], sems[j])
            accumulate(c, bufs[j])

            @pl.when(c + 2 < N_CHUNKS)
            def _():
                fire(c + 2, bufs[j], sems[j])

        return carry

    lax.fori_loop(0, N_CHUNKS // 2, chunk_body, 0)
    pltpu.sync_copy(out_v, out_hbm.at[pl.ds(wid * BAGS_PER_W, BAGS_PER_W)])


@functools.partial(
    pl.kernel,
    mesh=plsc.VectorSubcoreMesh(core_axis_name="c", subcore_axis_name="s"),
    out_type=jax.ShapeDtypeStruct((B, D), jnp.float32),
    compiler_params=pltpu.CompilerParams(use_tc_tiling_on_sc=False),
    scratch_types=[
        pltpu.VMEM((IDX_PER_W,), jnp.int32),
        pltpu.VMEM((CHUNK_IDX, D), jnp.float32),
        pltpu.VMEM((CHUNK_IDX, D), jnp.float32),
        pltpu.VMEM((BAGS_PER_W, D), jnp.float32),
        pltpu.SemaphoreType.DMA,
        pltpu.SemaphoreType.DMA,
    ],
)
def _ebag(idx_hbm, table_hbm, out_hbm, idx_v, rows0, rows1, out_v, sem0, sem1):
    _ebag_body(idx_hbm, table_hbm, out_hbm, idx_v, rows0, rows1, out_v, sem0, sem1)


def kernel(input, weight):
    idx = jnp.asarray(input, jnp.int32).reshape(-1)
    return _ebag(idx, weight)
